# baseline (speedup 1.0000x reference)
import functools

import jax
import jax.numpy as jnp
from jax import lax
from jax.experimental import pallas as pl
from jax.experimental.pallas import tpu as pltpu
from jax.experimental.pallas import tpu_sc as plsc

FEAT = 128
N_DATA = 100000
NCE_K = 4096
KD_T = 4.0
NCE_T = 0.05
MOM = 0.5
EPS = 1e-5
B = 64
KP = 4224
CHUNK = 128
NCHUNK = KP // CHUNK
ROWS_PER_TILE = 4000
NTILES = N_DATA // ROWS_PER_TILE

_f32 = jnp.float32
_i32 = jnp.int32


def _dotT(a, b):
    return lax.dot_general(a, b, (((1,), (1,)), ((), ())),
                           preferred_element_type=_f32)


def _embed_body(fs_ref, ft_ref,
                a_s, g_s, be_s, w1s, b1s, w2s, b2s, w3s, b3s,
                a_t, g_t, be_t, w1t, b1t, w2t, b2t, w3t, b3t,
                es_ref, et_ref):
    def one(f_ref, al, ga, be, w1, b1, w2, b2, w3, b3):
        x = f_ref[...]
        ss = jnp.sum(x * x, axis=1)
        emb = jnp.sqrt(ss + EPS) * al[...]
        msq = jnp.mean(emb * emb, axis=1, keepdims=True)
        nrm = ga[...] / jnp.sqrt(msq + EPS)
        gate = 1.0 + jnp.tanh(emb * nrm + be[...])
        h = jnp.mean(x * gate[:, None, :], axis=1)
        h = jnp.maximum(_dotT(h, w1[...]) + b1[...], 0.0)
        h = jnp.maximum(_dotT(h, w2[...]) + b2[...], 0.0)
        h = _dotT(h, w3[...]) + b3[...]
        return h / jnp.sqrt(jnp.sum(h * h, axis=1, keepdims=True))

    es_ref[...] = one(fs_ref, a_s, g_s, be_s, w1s, b1s, w2s, b2s, w3s, b3s)
    et_ref[...] = one(ft_ref, a_t, g_t, be_t, w1t, b1t, w2t, b2t, w3t, b3t)


def _idx_rows_body(ms_hbm, mt_hbm, idx_hbm, rs_hbm, rt_hbm,
                   idx_v, rs_v, rt_v, sem):
    wid = lax.axis_index("s") * 2 + lax.axis_index("c")

    @pl.when(wid == 0)
    def _():
        pltpu.sync_copy(idx_hbm, idx_v)
        pltpu.async_copy(ms_hbm.at[idx_v], rs_v, sem).wait()
        pltpu.async_copy(mt_hbm.at[idx_v], rt_v, sem).wait()
        pltpu.sync_copy(rs_v, rs_hbm)
        pltpu.sync_copy(rt_v, rt_hbm)


def _prep_body(rs_ref, rt_ref, es_ref, et_ref, ep_ref, cus_ref, cut_ref):
    es = es_ref[...]
    et = et_ref[...]
    u1 = rs_ref[...] * MOM + es * (1.0 - MOM)
    u1 = u1 / jnp.sqrt(jnp.sum(u1 * u1, axis=1, keepdims=True))
    u2 = rt_ref[...] * MOM + et * (1.0 - MOM)
    u2 = u2 / jnp.sqrt(jnp.sum(u2 * u2, axis=1, keepdims=True))
    ep = ep_ref[...]
    cus_ref[...] = _dotT(u1, ep)
    cut_ref[...] = _dotT(u2, ep)


def _score_body(ms_ref, mt_ref, ep_ref, cus_ref, cut_ref, idx_ref,
                ss_ref, st_ref):
    i = pl.program_id(0)
    a = _dotT(ms_ref[...], ep_ref[...])
    b = _dotT(mt_ref[...], ep_ref[...])
    rows = (lax.broadcasted_iota(_i32, (ROWS_PER_TILE, B), 0)
            + i * ROWS_PER_TILE)
    match = rows == idx_ref[0:1, :]
    jio = lax.broadcasted_iota(_i32, (ROWS_PER_TILE, B), 1)
    jm = jnp.max(jnp.where(match, jio, -1), axis=1, keepdims=True)
    has = jm >= 0
    onehot = ((jio == jm) & match).astype(_f32)
    corr_s = jnp.dot(onehot, cus_ref[...], preferred_element_type=_f32)
    corr_t = jnp.dot(onehot, cut_ref[...], preferred_element_type=_f32)
    lane = lax.broadcasted_iota(_i32, (ROWS_PER_TILE, FEAT), 1)
    even = (lane % 2) == 0
    ss_ref[...] = jnp.where(has & even, corr_s, a)
    st_ref[...] = jnp.where(has & (~even), corr_t, b)


def _pair_gather_body(ss2_hbm, st2_hbm, jidx_hbm, gs_hbm, gt_hbm,
                      idx_v, ra_v, rb_v, sem, nw):
    wid = lax.axis_index("s") * 2 + lax.axis_index("c")
    bpw = B // nw
    for off in range(bpw):
        bb = wid * bpw + off
        pltpu.sync_copy(jidx_hbm.at[bb], idx_v)
        descs = []
        for c in range(NCHUNK):
            descs.append(pltpu.async_copy(st2_hbm.at[idx_v.at[c]],
                                          ra_v.at[c], sem))
            descs.append(pltpu.async_copy(ss2_hbm.at[idx_v.at[c]],
                                          rb_v.at[c], sem))
        for dsc in descs:
            dsc.wait()
        pltpu.sync_copy(ra_v, gs_hbm.at[bb])
        pltpu.sync_copy(rb_v, gt_hbm.at[bb])


def _loss_body(gs1_ref, gs2_ref, gt1_ref, gt2_ref, out_ref):
    kmask = lax.broadcasted_iota(_i32, (B, KP), 1) < (NCE_K + 1)

    def mk_out(g):
        e = jnp.where(kmask, jnp.exp(g / NCE_T), 0.0)
        z = jnp.sum(e) / float(B * (NCE_K + 1)) * float(N_DATA)
        return e / z

    def logsm(y):
        lg = jnp.where(kmask, y / KD_T, -1e9)
        m = jnp.max(lg, axis=1, keepdims=True)
        shifted = lg - m
        return shifted - jnp.log(
            jnp.sum(jnp.exp(shifted), axis=1, keepdims=True))

    def kl(ys, yt):
        p_s = logsm(ys)
        lpt = logsm(yt)
        p_t = jnp.where(kmask, jnp.exp(lpt), 0.0)
        return jnp.sum(p_t * (lpt - p_s), axis=(0, 1),
                       keepdims=True) * (KD_T * KD_T / B)

    o_s1 = mk_out(gs1_ref[...])
    o_s2 = mk_out(gs2_ref[...])
    o_t1 = mk_out(gt1_ref[...])
    o_t2 = mk_out(gt2_ref[...])
    out_ref[...] = kl(o_s1, o_t1) + kl(o_s2, o_t2)


def kernel(f_s, f_t, params, mem_s, mem_t, idx, contrast_idx):
    ps, pt = params['s'], params['t']
    fs = jnp.transpose(f_s.reshape(B, FEAT, 49), (0, 2, 1))
    ft = jnp.transpose(f_t.reshape(B, FEAT, 49), (0, 2, 1))

    def flat(p, n):
        return p[n].reshape(1, FEAT)

    embed_args = [fs, ft]
    for p in (ps, pt):
        embed_args += [flat(p, 'alpha'), flat(p, 'gamma'), flat(p, 'beta'),
                       p['W1'], p['b1'].reshape(1, FEAT),
                       p['W2'], p['b2'].reshape(1, FEAT),
                       p['W3'], p['b3'].reshape(1, FEAT)]

    e_s, e_t = pl.pallas_call(
        _embed_body,
        out_shape=[jax.ShapeDtypeStruct((B, FEAT), _f32)] * 2,
    )(*embed_args)

    idx32 = idx.astype(_i32)
    cidx32 = contrast_idx.astype(_i32)

    rows_s = jnp.take(mem_s, idx32, axis=0)
    rows_t = jnp.take(mem_t, idx32, axis=0)

    e_perm = jnp.stack([e_s, e_t], axis=1).reshape(FEAT, FEAT)

    cus, cut = pl.pallas_call(
        _prep_body,
        out_shape=[jax.ShapeDtypeStruct((B, FEAT), _f32)] * 2,
    )(rows_s, rows_t, e_s, e_t, e_perm)

    a = jnp.dot(mem_s, e_perm.T, preferred_element_type=_f32)
    bm = jnp.dot(mem_t, e_perm.T, preferred_element_type=_f32)
    rows_iota = jnp.arange(N_DATA, dtype=_i32)[:, None]
    match = rows_iota == idx32[None, :]
    jio = jnp.arange(B, dtype=_i32)[None, :]
    jm = jnp.max(jnp.where(match, jio, -1), axis=1, keepdims=True)
    has = jm >= 0
    onehot = ((jio == jm) & match).astype(_f32)
    corr_s = jnp.dot(onehot, cus, preferred_element_type=_f32)
    corr_t = jnp.dot(onehot, cut, preferred_element_type=_f32)
    even = (jnp.arange(FEAT, dtype=_i32)[None, :] % 2) == 0
    s_s = jnp.where(has & even, corr_s, a)
    s_t = jnp.where(has & (~even), corr_t, bm)

    indices = jnp.concatenate([idx32[:, None], cidx32], axis=1)
    indices = jnp.pad(indices, ((0, 0), (0, KP - (NCE_K + 1))))
    jidx = (indices * (FEAT // 2)
            + jnp.arange(B, dtype=_i32)[:, None]).reshape(B, NCHUNK, CHUNK)

    ss2 = s_s.reshape(N_DATA * (FEAT // 2), 2)
    st2 = s_t.reshape(N_DATA * (FEAT // 2), 2)

    jflat = jidx.reshape(-1)
    gs_raw = jnp.take(st2, jflat, axis=0).reshape(B, NCHUNK, CHUNK, 2)
    gt_raw = jnp.take(ss2, jflat, axis=0).reshape(B, NCHUNK, CHUNK, 2)

    gs = gs_raw.reshape(B, KP, 2)
    gt = gt_raw.reshape(B, KP, 2)
    g_s1 = gs[:, :, 0]
    g_s2 = gs[:, :, 1]
    g_t2 = gt[:, :, 0]
    g_t1 = gt[:, :, 1]

    loss = pl.pallas_call(
        _loss_body,
        out_shape=jax.ShapeDtypeStruct((1, 1), _f32),
    )(g_s1, g_s2, g_t1, g_t2)
    return loss[0, 0]


# speedup vs baseline: 7.3643x; 7.3643x over previous
import functools

import jax
import jax.numpy as jnp
from jax import lax
from jax.experimental import pallas as pl
from jax.experimental.pallas import tpu as pltpu
from jax.experimental.pallas import tpu_sc as plsc

FEAT = 128
N_DATA = 100000
NCE_K = 4096
KD_T = 4.0
NCE_T = 0.05
MOM = 0.5
EPS = 1e-5
B = 64
KP = 4224
CHUNK = 128
NCHUNK = KP // CHUNK
ROWS_PER_TILE = 4000
NTILES = N_DATA // ROWS_PER_TILE

_f32 = jnp.float32
_i32 = jnp.int32


def _dotT(a, b):
    return lax.dot_general(a, b, (((1,), (1,)), ((), ())),
                           preferred_element_type=_f32)


def _embed_body(fs_ref, ft_ref,
                a_s, g_s, be_s, w1s, b1s, w2s, b2s, w3s, b3s,
                a_t, g_t, be_t, w1t, b1t, w2t, b2t, w3t, b3t,
                es_ref, et_ref):
    def one(f_ref, al, ga, be, w1, b1, w2, b2, w3, b3):
        x = f_ref[...]
        ss = jnp.sum(x * x, axis=1)
        emb = jnp.sqrt(ss + EPS) * al[...]
        msq = jnp.mean(emb * emb, axis=1, keepdims=True)
        nrm = ga[...] / jnp.sqrt(msq + EPS)
        gate = 1.0 + jnp.tanh(emb * nrm + be[...])
        h = jnp.mean(x * gate[:, None, :], axis=1)
        h = jnp.maximum(_dotT(h, w1[...]) + b1[...], 0.0)
        h = jnp.maximum(_dotT(h, w2[...]) + b2[...], 0.0)
        h = _dotT(h, w3[...]) + b3[...]
        return h / jnp.sqrt(jnp.sum(h * h, axis=1, keepdims=True))

    es_ref[...] = one(fs_ref, a_s, g_s, be_s, w1s, b1s, w2s, b2s, w3s, b3s)
    et_ref[...] = one(ft_ref, a_t, g_t, be_t, w1t, b1t, w2t, b2t, w3t, b3t)


def _idx_rows_body(ms_hbm, mt_hbm, idx_hbm, rs_hbm, rt_hbm,
                   idx_v, rs_v, rt_v, sem):
    wid = lax.axis_index("s") * 2 + lax.axis_index("c")

    @pl.when(wid == 0)
    def _():
        pltpu.sync_copy(idx_hbm, idx_v)
        pltpu.async_copy(ms_hbm.at[idx_v], rs_v, sem).wait()
        pltpu.async_copy(mt_hbm.at[idx_v], rt_v, sem).wait()
        pltpu.sync_copy(rs_v, rs_hbm)
        pltpu.sync_copy(rt_v, rt_hbm)


def _prep_body(rs_ref, rt_ref, es_ref, et_ref, ep_ref, cus_ref, cut_ref):
    es = es_ref[...]
    et = et_ref[...]
    u1 = rs_ref[...] * MOM + es * (1.0 - MOM)
    u1 = u1 / jnp.sqrt(jnp.sum(u1 * u1, axis=1, keepdims=True))
    u2 = rt_ref[...] * MOM + et * (1.0 - MOM)
    u2 = u2 / jnp.sqrt(jnp.sum(u2 * u2, axis=1, keepdims=True))
    ep = ep_ref[...]
    cus_ref[...] = _dotT(u1, ep)
    cut_ref[...] = _dotT(u2, ep)


def _score_body(ms_ref, mt_ref, ep_ref, cus_ref, cut_ref, idx_ref,
                ss_ref, st_ref):
    i = pl.program_id(0)
    a = _dotT(ms_ref[...], ep_ref[...])
    b = _dotT(mt_ref[...], ep_ref[...])
    rows = (lax.broadcasted_iota(_i32, (ROWS_PER_TILE, B), 0)
            + i * ROWS_PER_TILE)
    match = rows == idx_ref[0:1, :]
    jio = lax.broadcasted_iota(_i32, (ROWS_PER_TILE, B), 1)
    jm = jnp.max(jnp.where(match, jio, -1), axis=1, keepdims=True)
    has = jm >= 0
    onehot = ((jio == jm) & match).astype(_f32)
    corr_s = jnp.dot(onehot, cus_ref[...], preferred_element_type=_f32)
    corr_t = jnp.dot(onehot, cut_ref[...], preferred_element_type=_f32)
    lane = lax.broadcasted_iota(_i32, (ROWS_PER_TILE, FEAT), 1)
    even = (lane % 2) == 0
    ss_ref[...] = jnp.where(has & even, corr_s, a)
    st_ref[...] = jnp.where(has & (~even), corr_t, b)


def _pair_gather_body(ss2_hbm, st2_hbm, jidx_hbm, gs_hbm, gt_hbm,
                      idx_v, ra_v, rb_v, sem, nw):
    wid = lax.axis_index("s") * 2 + lax.axis_index("c")
    bpw = B // nw
    for off in range(bpw):
        bb = wid * bpw + off
        pltpu.sync_copy(jidx_hbm.at[bb], idx_v)
        descs = []
        for c in range(NCHUNK):
            descs.append(pltpu.async_copy(st2_hbm.at[idx_v.at[c]],
                                          ra_v.at[c], sem))
            descs.append(pltpu.async_copy(ss2_hbm.at[idx_v.at[c]],
                                          rb_v.at[c], sem))
        for dsc in descs:
            dsc.wait()
        pltpu.sync_copy(ra_v, gs_hbm.at[bb])
        pltpu.sync_copy(rb_v, gt_hbm.at[bb])


def _loss_body(gs1_ref, gs2_ref, gt1_ref, gt2_ref, out_ref):
    kmask = lax.broadcasted_iota(_i32, (B, KP), 1) < (NCE_K + 1)

    def mk_out(g):
        e = jnp.where(kmask, jnp.exp(g / NCE_T), 0.0)
        z = jnp.sum(e) / float(B * (NCE_K + 1)) * float(N_DATA)
        return e / z

    def logsm(y):
        lg = jnp.where(kmask, y / KD_T, -1e9)
        m = jnp.max(lg, axis=1, keepdims=True)
        shifted = lg - m
        return shifted - jnp.log(
            jnp.sum(jnp.exp(shifted), axis=1, keepdims=True))

    def kl(ys, yt):
        p_s = logsm(ys)
        lpt = logsm(yt)
        p_t = jnp.where(kmask, jnp.exp(lpt), 0.0)
        return jnp.sum(p_t * (lpt - p_s), axis=(0, 1),
                       keepdims=True) * (KD_T * KD_T / B)

    o_s1 = mk_out(gs1_ref[...])
    o_s2 = mk_out(gs2_ref[...])
    o_t1 = mk_out(gt1_ref[...])
    o_t2 = mk_out(gt2_ref[...])
    out_ref[...] = kl(o_s1, o_t1) + kl(o_s2, o_t2)


def kernel(f_s, f_t, params, mem_s, mem_t, idx, contrast_idx):
    ps, pt = params['s'], params['t']
    fs = jnp.transpose(f_s.reshape(B, FEAT, 49), (0, 2, 1))
    ft = jnp.transpose(f_t.reshape(B, FEAT, 49), (0, 2, 1))

    def flat(p, n):
        return p[n].reshape(1, FEAT)

    embed_args = [fs, ft]
    for p in (ps, pt):
        embed_args += [flat(p, 'alpha'), flat(p, 'gamma'), flat(p, 'beta'),
                       p['W1'], p['b1'].reshape(1, FEAT),
                       p['W2'], p['b2'].reshape(1, FEAT),
                       p['W3'], p['b3'].reshape(1, FEAT)]

    e_s, e_t = pl.pallas_call(
        _embed_body,
        out_shape=[jax.ShapeDtypeStruct((B, FEAT), _f32)] * 2,
    )(*embed_args)

    idx32 = idx.astype(_i32)
    cidx32 = contrast_idx.astype(_i32)

    rows_s = jnp.take(mem_s, idx32, axis=0)
    rows_t = jnp.take(mem_t, idx32, axis=0)

    e_perm = jnp.stack([e_s, e_t], axis=1).reshape(FEAT, FEAT)

    cus, cut = pl.pallas_call(
        _prep_body,
        out_shape=[jax.ShapeDtypeStruct((B, FEAT), _f32)] * 2,
    )(rows_s, rows_t, e_s, e_t, e_perm)

    a = jnp.dot(mem_s, e_perm.T, preferred_element_type=_f32)
    bm = jnp.dot(mem_t, e_perm.T, preferred_element_type=_f32)
    rows_iota = jnp.arange(N_DATA, dtype=_i32)[:, None]
    match = rows_iota == idx32[None, :]
    jio = jnp.arange(B, dtype=_i32)[None, :]
    jm = jnp.max(jnp.where(match, jio, -1), axis=1, keepdims=True)
    has = jm >= 0
    onehot = ((jio == jm) & match).astype(_f32)
    corr_s = jnp.dot(onehot, cus, preferred_element_type=_f32)
    corr_t = jnp.dot(onehot, cut, preferred_element_type=_f32)
    even = (jnp.arange(FEAT, dtype=_i32)[None, :] % 2) == 0
    s_s = jnp.where(has & even, corr_s, a)
    s_t = jnp.where(has & (~even), corr_t, bm)

    indices = jnp.concatenate([idx32[:, None], cidx32], axis=1)
    indices = jnp.pad(indices, ((0, 0), (0, KP - (NCE_K + 1))))
    jidx = (indices * (FEAT // 2)
            + jnp.arange(B, dtype=_i32)[:, None]).reshape(B, NCHUNK, CHUNK)

    ss2 = s_s.reshape(N_DATA * (FEAT // 2), 2)
    st2 = s_t.reshape(N_DATA * (FEAT // 2), 2)

    del jidx
    gs_raw = st2[:B * KP].reshape(B, NCHUNK, CHUNK, 2)
    gt_raw = ss2[:B * KP].reshape(B, NCHUNK, CHUNK, 2)

    gs = gs_raw.reshape(B, KP, 2)
    gt = gt_raw.reshape(B, KP, 2)
    g_s1 = gs[:, :, 0]
    g_s2 = gs[:, :, 1]
    g_t2 = gt[:, :, 0]
    g_t1 = gt[:, :, 1]

    loss = pl.pallas_call(
        _loss_body,
        out_shape=jax.ShapeDtypeStruct((1, 1), _f32),
    )(g_s1, g_s2, g_t1, g_t2)
    return loss[0, 0]


# speedup vs baseline: 11.8380x; 1.6075x over previous
import functools

import jax
import jax.numpy as jnp
from jax import lax
from jax.experimental import pallas as pl
from jax.experimental.pallas import tpu as pltpu
from jax.experimental.pallas import tpu_sc as plsc

FEAT = 128
N_DATA = 100000
NCE_K = 4096
KD_T = 4.0
NCE_T = 0.05
MOM = 0.5
EPS = 1e-5
B = 64
KP = 4224
CHUNK = 128
NCHUNK = KP // CHUNK
ROWS_PER_TILE = 4000
NTILES = N_DATA // ROWS_PER_TILE

_f32 = jnp.float32
_i32 = jnp.int32


def _dotT(a, b):
    return lax.dot_general(a, b, (((1,), (1,)), ((), ())),
                           preferred_element_type=_f32)


def _embed_body(fs_ref, ft_ref,
                a_s, g_s, be_s, w1s, b1s, w2s, b2s, w3s, b3s,
                a_t, g_t, be_t, w1t, b1t, w2t, b2t, w3t, b3t,
                es_ref, et_ref):
    def one(f_ref, al, ga, be, w1, b1, w2, b2, w3, b3):
        x = f_ref[...]
        ss = jnp.sum(x * x, axis=1)
        emb = jnp.sqrt(ss + EPS) * al[...]
        msq = jnp.mean(emb * emb, axis=1, keepdims=True)
        nrm = ga[...] / jnp.sqrt(msq + EPS)
        gate = 1.0 + jnp.tanh(emb * nrm + be[...])
        h = jnp.mean(x * gate[:, None, :], axis=1)
        h = jnp.maximum(_dotT(h, w1[...]) + b1[...], 0.0)
        h = jnp.maximum(_dotT(h, w2[...]) + b2[...], 0.0)
        h = _dotT(h, w3[...]) + b3[...]
        return h / jnp.sqrt(jnp.sum(h * h, axis=1, keepdims=True))

    es_ref[...] = one(fs_ref, a_s, g_s, be_s, w1s, b1s, w2s, b2s, w3s, b3s)
    et_ref[...] = one(ft_ref, a_t, g_t, be_t, w1t, b1t, w2t, b2t, w3t, b3t)


def _idx_rows_body(ms_hbm, mt_hbm, idx_hbm, rs_hbm, rt_hbm,
                   idx_v, rs_v, rt_v, sem):
    wid = lax.axis_index("s") * 2 + lax.axis_index("c")

    @pl.when(wid == 0)
    def _():
        pltpu.sync_copy(idx_hbm, idx_v)
        pltpu.async_copy(ms_hbm.at[idx_v], rs_v, sem).wait()
        pltpu.async_copy(mt_hbm.at[idx_v], rt_v, sem).wait()
        pltpu.sync_copy(rs_v, rs_hbm)
        pltpu.sync_copy(rt_v, rt_hbm)


def _prep_body(rs_ref, rt_ref, es_ref, et_ref, ep_ref, cus_ref, cut_ref):
    es = es_ref[...]
    et = et_ref[...]
    u1 = rs_ref[...] * MOM + es * (1.0 - MOM)
    u1 = u1 / jnp.sqrt(jnp.sum(u1 * u1, axis=1, keepdims=True))
    u2 = rt_ref[...] * MOM + et * (1.0 - MOM)
    u2 = u2 / jnp.sqrt(jnp.sum(u2 * u2, axis=1, keepdims=True))
    ep = ep_ref[...]
    cus_ref[...] = _dotT(u1, ep)
    cut_ref[...] = _dotT(u2, ep)


def _score_body(ms_ref, mt_ref, ep_ref, cus_ref, cut_ref, idx_ref,
                ss_ref, st_ref):
    i = pl.program_id(0)
    a = _dotT(ms_ref[...], ep_ref[...])
    b = _dotT(mt_ref[...], ep_ref[...])
    rows = (lax.broadcasted_iota(_i32, (ROWS_PER_TILE, B), 0)
            + i * ROWS_PER_TILE)
    match = rows == idx_ref[0:1, :]
    jio = lax.broadcasted_iota(_i32, (ROWS_PER_TILE, B), 1)
    jm = jnp.max(jnp.where(match, jio, -1), axis=1, keepdims=True)
    has = jm >= 0
    onehot = ((jio == jm) & match).astype(_f32)
    corr_s = jnp.dot(onehot, cus_ref[...], preferred_element_type=_f32)
    corr_t = jnp.dot(onehot, cut_ref[...], preferred_element_type=_f32)
    lane = lax.broadcasted_iota(_i32, (ROWS_PER_TILE, FEAT), 1)
    even = (lane % 2) == 0
    ss_ref[...] = jnp.where(has & even, corr_s, a)
    st_ref[...] = jnp.where(has & (~even), corr_t, b)


NBUF = 3
NGROUP = NCHUNK // NBUF


def _row_gather_body(ss_hbm, st_hbm, jidx_hbm,
                     g1_hbm, g2_hbm, g3_hbm, g4_hbm,
                     idx_v, rs0_v, rs1_v, rs2_v, rt0_v, rt1_v, rt2_v,
                     o1_v, o2_v, o3_v, o4_v, sem0, sem1, sem2, nw):
    wid = lax.axis_index("s") * 2 + lax.axis_index("c")
    bpw = B // nw
    rings_s = [rs0_v, rs1_v, rs2_v]
    rings_t = [rt0_v, rt1_v, rt2_v]
    sems = [sem0, sem1, sem2]

    def fire(c, j):
        pltpu.async_copy(ss_hbm.at[idx_v.at[c]], rings_s[j], sems[j])
        pltpu.async_copy(st_hbm.at[idx_v.at[c]], rings_t[j], sems[j])

    def drain(c, j):
        pltpu.make_async_copy(ss_hbm.at[idx_v.at[c]],
                              rings_s[j], sems[j]).wait()
        pltpu.make_async_copy(st_hbm.at[idx_v.at[c]],
                              rings_t[j], sems[j]).wait()

    for off in range(bpw):
        bb = wid * bpw + off
        plan = [(rings_t, 2 * bb, o1_v), (rings_t, 2 * bb + 1, o2_v),
                (rings_s, 2 * bb + 1, o3_v), (rings_s, 2 * bb, o4_v)]
        pltpu.sync_copy(jidx_hbm.at[bb], idx_v)
        for j in range(NBUF):
            fire(j, j)

        def group_step(g, _):
            base = g * NBUF
            for j in range(NBUF):
                c = base + j
                drain(c, j)
                for sub in range(8):
                    rows16 = lax.iota(_i32, 16) + (sub * 16)
                    for rg, col, ov in plan:
                        col16 = jnp.full((16,), col, _i32)
                        vals = plsc.load_gather(rg[j], [rows16, col16])
                        ov[c, pl.ds(sub * 16, 16)] = vals

                @pl.when(c + NBUF < NCHUNK)
                def _():
                    fire(c + NBUF, j)
            return 0

        lax.fori_loop(0, NGROUP, group_step, 0)
        pltpu.sync_copy(o1_v, g1_hbm.at[bb])
        pltpu.sync_copy(o2_v, g2_hbm.at[bb])
        pltpu.sync_copy(o3_v, g3_hbm.at[bb])
        pltpu.sync_copy(o4_v, g4_hbm.at[bb])


def _loss_body(gs1_ref, gs2_ref, gt1_ref, gt2_ref, out_ref):
    kmask = lax.broadcasted_iota(_i32, (B, KP), 1) < (NCE_K + 1)

    def mk_out(g):
        e = jnp.where(kmask, jnp.exp(g / NCE_T), 0.0)
        z = jnp.sum(e) / float(B * (NCE_K + 1)) * float(N_DATA)
        return e / z

    def logsm(y):
        lg = jnp.where(kmask, y / KD_T, -1e9)
        m = jnp.max(lg, axis=1, keepdims=True)
        shifted = lg - m
        return shifted - jnp.log(
            jnp.sum(jnp.exp(shifted), axis=1, keepdims=True))

    def kl(ys, yt):
        p_s = logsm(ys)
        lpt = logsm(yt)
        p_t = jnp.where(kmask, jnp.exp(lpt), 0.0)
        return jnp.sum(p_t * (lpt - p_s), axis=(0, 1),
                       keepdims=True) * (KD_T * KD_T / B)

    o_s1 = mk_out(gs1_ref[...])
    o_s2 = mk_out(gs2_ref[...])
    o_t1 = mk_out(gt1_ref[...])
    o_t2 = mk_out(gt2_ref[...])
    out_ref[...] = kl(o_s1, o_t1) + kl(o_s2, o_t2)


def kernel(f_s, f_t, params, mem_s, mem_t, idx, contrast_idx):
    ps, pt = params['s'], params['t']
    fs = jnp.transpose(f_s.reshape(B, FEAT, 49), (0, 2, 1))
    ft = jnp.transpose(f_t.reshape(B, FEAT, 49), (0, 2, 1))

    def flat(p, n):
        return p[n].reshape(1, FEAT)

    embed_args = [fs, ft]
    for p in (ps, pt):
        embed_args += [flat(p, 'alpha'), flat(p, 'gamma'), flat(p, 'beta'),
                       p['W1'], p['b1'].reshape(1, FEAT),
                       p['W2'], p['b2'].reshape(1, FEAT),
                       p['W3'], p['b3'].reshape(1, FEAT)]

    e_s, e_t = pl.pallas_call(
        _embed_body,
        out_shape=[jax.ShapeDtypeStruct((B, FEAT), _f32)] * 2,
    )(*embed_args)

    idx32 = idx.astype(_i32)
    cidx32 = contrast_idx.astype(_i32)

    mesh = plsc.VectorSubcoreMesh(core_axis_name="c", subcore_axis_name="s")
    rows_s, rows_t = pl.kernel(
        _idx_rows_body,
        out_type=[jax.ShapeDtypeStruct((B, FEAT), _f32)] * 2,
        mesh=mesh,
        scratch_types=[pltpu.VMEM((B,), _i32),
                       pltpu.VMEM((B, FEAT), _f32),
                       pltpu.VMEM((B, FEAT), _f32),
                       pltpu.SemaphoreType.DMA],
    )(mem_s, mem_t, idx32)

    e_perm = jnp.stack([e_s, e_t], axis=1).reshape(FEAT, FEAT)

    cus, cut = pl.pallas_call(
        _prep_body,
        out_shape=[jax.ShapeDtypeStruct((B, FEAT), _f32)] * 2,
    )(rows_s, rows_t, e_s, e_t, e_perm)

    idx_b = jnp.broadcast_to(idx32[None, :], (8, B))
    c_mat = pl.pallas_call(
        _score_body,
        grid=(NTILES,),
        in_specs=[
            pl.BlockSpec((ROWS_PER_TILE, FEAT), lambda i: (i, 0)),
            pl.BlockSpec((ROWS_PER_TILE, FEAT), lambda i: (i, 0)),
            pl.BlockSpec((FEAT, FEAT), lambda i: (0, 0)),
            pl.BlockSpec((B, FEAT), lambda i: (0, 0)),
            pl.BlockSpec((B, FEAT), lambda i: (0, 0)),
            pl.BlockSpec((8, B), lambda i: (0, 0)),
        ],
        out_specs=[
            pl.BlockSpec((ROWS_PER_TILE, FEAT), lambda i: (i, 0)),
            pl.BlockSpec((ROWS_PER_TILE, FEAT), lambda i: (i, 0)),
        ],
        out_shape=[jax.ShapeDtypeStruct((N_DATA, FEAT), _f32)] * 2,
    )(mem_s, mem_t, e_perm, cus, cut, idx_b)
    s_s, s_t = c_mat

    indices = jnp.concatenate([idx32[:, None], cidx32], axis=1)
    indices = jnp.pad(indices, ((0, 0), (0, KP - (NCE_K + 1))))
    jidx = indices.reshape(B, NCHUNK, CHUNK)

    info = plsc.get_sparse_core_info()
    nw = info.num_cores * info.num_subcores
    g_s1, g_s2, g_t1, g_t2 = pl.kernel(
        functools.partial(_row_gather_body, nw=nw),
        out_type=[jax.ShapeDtypeStruct((B, NCHUNK, CHUNK), _f32)] * 4,
        mesh=mesh,
        scratch_types=[pltpu.VMEM((NCHUNK, CHUNK), _i32)]
        + [pltpu.VMEM((CHUNK, FEAT), _f32)] * (2 * NBUF)
        + [pltpu.VMEM((NCHUNK, CHUNK), _f32)] * 4
        + [pltpu.SemaphoreType.DMA] * NBUF,
        compiler_params=pltpu.CompilerParams(needs_layout_passes=False),
    )(s_s, s_t, jidx)

    loss = pl.pallas_call(
        _loss_body,
        out_shape=jax.ShapeDtypeStruct((1, 1), _f32),
    )(g_s1.reshape(B, KP), g_s2.reshape(B, KP),
      g_t1.reshape(B, KP), g_t2.reshape(B, KP))
    return loss[0, 0]
